# Initial kernel scaffold; baseline (speedup 1.0000x reference)
import jax
import jax.numpy as jnp
from jax.experimental import pallas as pl


def kernel(X, W):
    raise NotImplementedError("write your pallas kernel here")


# speedup vs baseline: 2.1166x; 2.1166x over previous
import functools

import jax
import jax.numpy as jnp
import numpy as np
from jax import lax
from jax.experimental import pallas as pl
from jax.experimental.pallas import tpu as pltpu
from jax.experimental.pallas import tpu_sc as plsc

_D = 128
_L = 16
_NC, _NS = 2, 16
_NW = _NC * _NS
_CH = 128


def _pos_table(max_len: int) -> jnp.ndarray:
    pos = np.arange(max_len, dtype=np.float32).reshape(-1, 1)
    div = np.power(10000.0, np.arange(0, _D, 2, dtype=np.float32) / _D)
    x = pos / div
    p = np.zeros((max_len, _D), np.float32)
    p[:, 0::2] = np.sin(x)
    p[:, 1::2] = np.cos(x)
    return jnp.asarray(p)


@functools.cache
def _emb_call(rows: int, seq: int):
    rpw = rows // _NW
    nch = rpw // _CH
    sper = seq // _CH
    mesh = plsc.VectorSubcoreMesh(
        core_axis_name="c", subcore_axis_name="s",
        num_cores=_NC, num_subcores=_NS)

    @functools.partial(
        pl.kernel,
        mesh=mesh,
        out_type=jax.ShapeDtypeStruct((rows, _D), jnp.float32),
        scratch_types=[
            pltpu.VMEM((_CH,), jnp.int32),
            pltpu.VMEM((_CH, _D), jnp.float32),
            pltpu.VMEM((seq, _D), jnp.float32),
            pltpu.SemaphoreType.DMA,
        ],
    )
    def k(x_hbm, w_hbm, p_hbm, out_hbm, idx_v, rows_v, p_v, gsem):
        wid = lax.axis_index("s") * _NC + lax.axis_index("c")
        base = wid * rpw
        pltpu.sync_copy(p_hbm, p_v)

        def chunk_body(c, carry):
            g0 = base + c * _CH
            s0 = lax.rem(c, sper) * _CH
            pltpu.sync_copy(x_hbm.at[pl.ds(g0, _CH)], idx_v)
            pltpu.async_copy(w_hbm.at[idx_v], rows_v, gsem).wait()

            def add_row(r, carry2):
                s = s0 + r
                for j in range(_D // _L):
                    pv = p_v[s, pl.ds(j * _L, _L)]
                    plsc.addupdate(rows_v.at[r, pl.ds(j * _L, _L)], pv)
                return carry2

            lax.fori_loop(0, _CH, add_row, 0, unroll=False)
            pltpu.sync_copy(rows_v, out_hbm.at[pl.ds(g0, _CH)])
            return carry

        lax.fori_loop(0, nch, chunk_body, 0, unroll=False)

    return k


def kernel(X, W):
    b, s = X.shape
    rows = b * s
    x_flat = X.reshape(rows)
    p = _pos_table(s)
    out = _emb_call(rows, s)(x_flat, W, p)
    return out.reshape(b, s, _D)


# speedup vs baseline: 2.8933x; 1.3669x over previous
import functools

import jax
import jax.numpy as jnp
import numpy as np
from jax import lax
from jax.experimental import pallas as pl
from jax.experimental.pallas import tpu as pltpu
from jax.experimental.pallas import tpu_sc as plsc

_D = 128
_L = 16
_NC, _NS = 2, 16
_NW = _NC * _NS
_CH = 128


def _pos_table(max_len: int) -> jnp.ndarray:
    pos = np.arange(max_len, dtype=np.float32).reshape(-1, 1)
    div = np.power(10000.0, np.arange(0, _D, 2, dtype=np.float32) / _D)
    x = pos / div
    p = np.zeros((max_len, _D), np.float32)
    p[:, 0::2] = np.sin(x)
    p[:, 1::2] = np.cos(x)
    return jnp.asarray(p)


@functools.cache
def _emb_call(rows: int, seq: int):
    rpw = rows // _NW
    nch = rpw // _CH
    sper = seq // _CH
    mesh = plsc.VectorSubcoreMesh(
        core_axis_name="c", subcore_axis_name="s",
        num_cores=_NC, num_subcores=_NS)

    @functools.partial(
        pl.kernel,
        mesh=mesh,
        out_type=jax.ShapeDtypeStruct((rows, _D), jnp.float32),
        scratch_types=[
            pltpu.VMEM((nch, _CH), jnp.int32),
            pltpu.VMEM((2, _CH, _D), jnp.float32),
            pltpu.VMEM((seq, _D), jnp.float32),
            pltpu.SemaphoreType.DMA,
            pltpu.SemaphoreType.DMA,
        ],
    )
    def k(x_hbm, w_hbm, p_hbm, out_hbm, idx_all, rows_v, p_v, gsem0, gsem1):
        wid = lax.axis_index("s") * _NC + lax.axis_index("c")
        base = wid * rpw
        gsems = (gsem0, gsem1)
        pltpu.sync_copy(p_hbm, p_v)
        pltpu.sync_copy(x_hbm.at[pl.ds(wid * nch, nch)], idx_all)

        def issue_gather(c, b):
            pltpu.async_copy(w_hbm.at[idx_all.at[c]], rows_v.at[b], gsems[b])

        def wait_gather(c, b):
            pltpu.make_async_copy(
                w_hbm.at[idx_all.at[c]], rows_v.at[b], gsems[b]).wait()

        issue_gather(0, 0)

        def pair_body(cc, carry):
            for b in range(2):
                c = cc * 2 + b

                @pl.when(c + 1 < nch)
                def _():
                    issue_gather(c + 1, 1 - b)

                wait_gather(c, b)
                s0 = lax.rem(c, sper) * _CH

                def add_row(r, carry2):
                    s = s0 + r
                    for j in range(_D // _L):
                        pv = p_v[s, pl.ds(j * _L, _L)]
                        plsc.addupdate(rows_v.at[b, r, pl.ds(j * _L, _L)], pv)
                    return carry2

                lax.fori_loop(0, _CH, add_row, 0, unroll=False)
                g0 = base + c * _CH
                pltpu.sync_copy(rows_v.at[b], out_hbm.at[pl.ds(g0, _CH)])
            return carry

        lax.fori_loop(0, nch // 2, pair_body, 0, unroll=False)

    return k


def kernel(X, W):
    b, s = X.shape
    rows = b * s
    x2d = X.reshape(rows // _CH, _CH)
    p = _pos_table(s)
    out = _emb_call(rows, s)(x2d, W, p)
    return out.reshape(b, s, _D)


# speedup vs baseline: 2.9827x; 1.0309x over previous
import functools

import jax
import jax.numpy as jnp
import numpy as np
from jax import lax
from jax.experimental import pallas as pl
from jax.experimental.pallas import tpu as pltpu
from jax.experimental.pallas import tpu_sc as plsc

_D = 128
_L = 16
_NC, _NS = 2, 16
_NW = _NC * _NS
_CH = 128


def _pos_table(max_len: int) -> jnp.ndarray:
    pos = np.arange(max_len, dtype=np.float32).reshape(-1, 1)
    div = np.power(10000.0, np.arange(0, _D, 2, dtype=np.float32) / _D)
    x = pos / div
    p = np.zeros((max_len, _D), np.float32)
    p[:, 0::2] = np.sin(x)
    p[:, 1::2] = np.cos(x)
    return jnp.asarray(p)


@functools.cache
def _emb_call(rows: int, seq: int):
    rpw = rows // _NW
    nch = rpw // _CH
    sper = seq // _CH
    mesh = plsc.VectorSubcoreMesh(
        core_axis_name="c", subcore_axis_name="s",
        num_cores=_NC, num_subcores=_NS)

    @functools.partial(
        pl.kernel,
        mesh=mesh,
        out_type=jax.ShapeDtypeStruct((rows, _D), jnp.float32),
        scratch_types=[
            pltpu.VMEM((nch, _CH), jnp.int32),
            pltpu.VMEM((2, _CH, _D), jnp.float32),
            pltpu.VMEM((seq, _D), jnp.float32),
            pltpu.SemaphoreType.DMA,
            pltpu.SemaphoreType.DMA,
        ],
    )
    def k(x_hbm, w_hbm, p_hbm, out_hbm, idx_all, rows_v, p_v, gsem0, gsem1):
        wid = lax.axis_index("s") * _NC + lax.axis_index("c")
        base = wid * rpw
        gsems = (gsem0, gsem1)
        pltpu.sync_copy(p_hbm, p_v)
        pltpu.sync_copy(x_hbm.at[pl.ds(wid * nch, nch)], idx_all)

        def issue_gather(c, b):
            pltpu.async_copy(w_hbm.at[idx_all.at[c]], rows_v.at[b], gsems[b])

        def wait_gather(c, b):
            pltpu.make_async_copy(
                w_hbm.at[idx_all.at[c]], rows_v.at[b], gsems[b]).wait()

        issue_gather(0, 0)

        def pair_body(cc, carry):
            for b in range(2):
                c = cc * 2 + b

                @pl.when(c + 1 < nch)
                def _():
                    issue_gather(c + 1, 1 - b)

                wait_gather(c, b)
                s0 = lax.rem(c, sper) * _CH

                def add_row(r, carry2):
                    s = s0 + r
                    for j in range(_D // _L):
                        pv = p_v[s, pl.ds(j * _L, _L)]
                        plsc.addupdate(rows_v.at[b, r, pl.ds(j * _L, _L)], pv)
                    return carry2

                lax.fori_loop(0, _CH, add_row, 0, unroll=8)
                g0 = base + c * _CH
                pltpu.sync_copy(rows_v.at[b], out_hbm.at[pl.ds(g0, _CH)])
            return carry

        lax.fori_loop(0, nch // 2, pair_body, 0, unroll=False)

    return k


def kernel(X, W):
    b, s = X.shape
    rows = b * s
    x2d = X.reshape(rows // _CH, _CH)
    p = _pos_table(s)
    out = _emb_call(rows, s)(x2d, W, p)
    return out.reshape(b, s, _D)


# speedup vs baseline: 4.1660x; 1.3968x over previous
import functools

import jax
import jax.numpy as jnp
import numpy as np
from jax import lax
from jax.experimental import pallas as pl
from jax.experimental.pallas import tpu as pltpu
from jax.experimental.pallas import tpu_sc as plsc

_D = 128
_L = 16
_NC, _NS = 2, 16
_NW = _NC * _NS
_CH = 128


def _pos_table(max_len: int) -> jnp.ndarray:
    pos = np.arange(max_len, dtype=np.float32).reshape(-1, 1)
    div = np.power(10000.0, np.arange(0, _D, 2, dtype=np.float32) / _D)
    x = pos / div
    p = np.zeros((max_len, _D), np.float32)
    p[:, 0::2] = np.sin(x)
    p[:, 1::2] = np.cos(x)
    return jnp.asarray(p)


@functools.cache
def _emb_call(rows: int, seq: int):
    rpw = rows // _NW
    nch = rpw // _CH
    sper = seq // _CH
    mesh = plsc.VectorSubcoreMesh(
        core_axis_name="c", subcore_axis_name="s",
        num_cores=_NC, num_subcores=_NS)

    @functools.partial(
        pl.kernel,
        mesh=mesh,
        out_type=jax.ShapeDtypeStruct((rows, _D), jnp.float32),
        scratch_types=[
            pltpu.VMEM((nch, _CH), jnp.int32),
            pltpu.VMEM((2, _CH, _D), jnp.float32),
            pltpu.SemaphoreType.DMA,
            pltpu.SemaphoreType.DMA,
            pltpu.SemaphoreType.DMA,
            pltpu.SemaphoreType.DMA,
        ],
    )
    def k(x_hbm, w_hbm, p_hbm, out_hbm, idx_all, rows_v,
          psem0, psem1, gsem0, gsem1):
        wid = lax.axis_index("s") * _NC + lax.axis_index("c")
        base = wid * rpw
        psems = (psem0, psem1)
        gsems = (gsem0, gsem1)
        pltpu.sync_copy(x_hbm.at[pl.ds(wid * nch, nch)], idx_all)

        def s0_of(c):
            return lax.rem(c, sper) * _CH

        def prefill(c, b, wait=False):
            if wait:
                pltpu.make_async_copy(
                    p_hbm.at[pl.ds(s0_of(c), _CH)], rows_v.at[b],
                    psems[b]).wait()
            else:
                pltpu.async_copy(
                    p_hbm.at[pl.ds(s0_of(c), _CH)], rows_v.at[b], psems[b])

        def gather_add(c, b, wait=False):
            if wait:
                pltpu.make_async_copy(
                    w_hbm.at[idx_all.at[c]], rows_v.at[b], gsems[b]).wait()
            else:
                pltpu.async_copy(
                    w_hbm.at[idx_all.at[c]], rows_v.at[b], gsems[b], add=True)

        prefill(0, 0)
        prefill(1, 1)
        prefill(0, 0, wait=True)
        gather_add(0, 0)

        def pair_body(cc, carry):
            for b in range(2):
                c = cc * 2 + b

                @pl.when(c + 1 < nch)
                def _():
                    prefill(c + 1, 1 - b, wait=True)
                    gather_add(c + 1, 1 - b)

                gather_add(c, b, wait=True)
                pltpu.sync_copy(
                    rows_v.at[b], out_hbm.at[pl.ds(base + c * _CH, _CH)])

                @pl.when(c + 2 < nch)
                def _():
                    prefill(c + 2, b)
            return carry

        lax.fori_loop(0, nch // 2, pair_body, 0, unroll=False)

    return k


def kernel(X, W):
    b, s = X.shape
    rows = b * s
    x2d = X.reshape(rows // _CH, _CH)
    p = _pos_table(s)
    out = _emb_call(rows, s)(x2d, W, p)
    return out.reshape(b, s, _D)
